# Initial kernel scaffold; baseline (speedup 1.0000x reference)
import functools

import jax
import jax.numpy as jnp
from jax import lax
from jax.experimental import pallas as pl
from jax.experimental.pallas import tpu as pltpu
from jax.experimental.pallas import tpu_sc as plsc

VOCAB = 1000000
EMBED_DIM = 32
PADDING_IDX = 0

NUM_CORES = 2
NUM_SUBCORES = 16
NUM_WORKERS = NUM_CORES * NUM_SUBCORES

BATCH, SEQ = 4096, 200
N = BATCH * SEQ
PER_WORKER = N // NUM_WORKERS
CHUNK = 128
NCHUNK = PER_WORKER // CHUNK


def _emb_body(idx_hbm, w_hbm, out_hbm, idx_v, rows_v, sem):
    wid = lax.axis_index("s") * NUM_CORES + lax.axis_index("c")
    cbase = wid * NCHUNK
    rbase = wid * PER_WORKER

    pltpu.sync_copy(idx_hbm.at[pl.ds(cbase, NCHUNK)], idx_v)

    zeros16 = jnp.zeros((16,), jnp.float32)

    def chunk_step(g, carry):
        pltpu.async_copy(w_hbm.at[idx_v.at[g]], rows_v, sem).wait()

        mn = idx_v[g, pl.ds(0, 16)]
        for l in range(1, CHUNK // 16):
            mn = jnp.minimum(mn, idx_v[g, pl.ds(l * 16, 16)])

        @pl.when(jnp.min(mn) == PADDING_IDX)
        def _fixup():
            def zrow(r, c):
                @pl.when(idx_v[g, r] == PADDING_IDX)
                def _z():
                    rows_v[r, pl.ds(0, 16)] = zeros16
                    rows_v[r, pl.ds(16, 16)] = zeros16

                return c

            lax.fori_loop(0, CHUNK, zrow, 0)

        pltpu.sync_copy(rows_v, out_hbm.at[pl.ds(rbase + g * CHUNK, CHUNK)])
        return carry

    lax.fori_loop(0, NCHUNK, chunk_step, 0)


@jax.jit
def _emb_call(idx2d, weight):
    mesh = plsc.VectorSubcoreMesh(core_axis_name="c", subcore_axis_name="s")
    fn = functools.partial(
        pl.kernel,
        mesh=mesh,
        out_type=jax.ShapeDtypeStruct((N, EMBED_DIM), jnp.float32),
        scratch_types=[
            pltpu.VMEM((NCHUNK, CHUNK), jnp.int32),
            pltpu.VMEM((CHUNK, EMBED_DIM), jnp.float32),
            pltpu.SemaphoreType.DMA,
        ],
    )(_emb_body)
    return fn(idx2d, weight)


def kernel(input_ids, weight):
    idx2d = input_ids.astype(jnp.int32).reshape(NUM_WORKERS * NCHUNK, CHUNK)
    out = _emb_call(idx2d, weight)
    return out.reshape(BATCH, SEQ, EMBED_DIM)


# speedup vs baseline: 1.3001x; 1.3001x over previous
import functools

import jax
import jax.numpy as jnp
from jax import lax
from jax.experimental import pallas as pl
from jax.experimental.pallas import tpu as pltpu
from jax.experimental.pallas import tpu_sc as plsc

VOCAB = 1000000
EMBED_DIM = 32
PADDING_IDX = 0

NUM_CORES = 2
NUM_SUBCORES = 16
NUM_WORKERS = NUM_CORES * NUM_SUBCORES

BATCH, SEQ = 4096, 200
N = BATCH * SEQ
PER_WORKER = N // NUM_WORKERS
CHUNK = 128
NCHUNK = PER_WORKER // CHUNK


def _emb_body(idx_hbm, w_hbm, out_hbm, idx_v, rows_v, sem):
    wid = lax.axis_index("s") * NUM_CORES + lax.axis_index("c")
    cbase = wid * NCHUNK
    rbase = wid * PER_WORKER

    pltpu.sync_copy(idx_hbm.at[pl.ds(cbase, NCHUNK)], idx_v)

    zeros16 = jnp.zeros((16,), jnp.float32)

    def chunk_step(g, carry):
        pltpu.async_copy(w_hbm.at[idx_v.at[g]], rows_v, sem).wait()

        macc = idx_v[g, pl.ds(0, 16)] == PADDING_IDX
        for l in range(1, CHUNK // 16):
            macc = jnp.logical_or(macc, idx_v[g, pl.ds(l * 16, 16)] == PADDING_IDX)
        any_pad = jnp.any(macc)

        @pl.when(any_pad)
        def _fixup():
            for gi in range(CHUNK // 16):
                iv = idx_v[g, pl.ds(gi * 16, 16)]
                m = iv == PADDING_IDX

                @pl.when(jnp.any(m))
                def _zgroup():
                    rows_idx = lax.iota(jnp.int32, 16) + gi * 16
                    for col in range(EMBED_DIM):
                        plsc.store_scatter(
                            rows_v,
                            [rows_idx, jnp.full((16,), col, jnp.int32)],
                            zeros16,
                            mask=m,
                        )

        pltpu.sync_copy(rows_v, out_hbm.at[pl.ds(rbase + g * CHUNK, CHUNK)])
        return carry

    lax.fori_loop(0, NCHUNK, chunk_step, 0)


@jax.jit
def _emb_call(idx2d, weight):
    mesh = plsc.VectorSubcoreMesh(core_axis_name="c", subcore_axis_name="s")
    fn = functools.partial(
        pl.kernel,
        mesh=mesh,
        out_type=jax.ShapeDtypeStruct((N, EMBED_DIM), jnp.float32),
        scratch_types=[
            pltpu.VMEM((NCHUNK, CHUNK), jnp.int32),
            pltpu.VMEM((CHUNK, EMBED_DIM), jnp.float32),
            pltpu.SemaphoreType.DMA,
        ],
        compiler_params=pltpu.CompilerParams(
            needs_layout_passes=False, use_tc_tiling_on_sc=False
        ),
    )(_emb_body)
    return fn(idx2d, weight)


def kernel(input_ids, weight):
    idx2d = input_ids.astype(jnp.int32).reshape(NUM_WORKERS * NCHUNK, CHUNK)
    out = _emb_call(idx2d, weight)
    return out.reshape(BATCH, SEQ, EMBED_DIM)


# speedup vs baseline: 1.4929x; 1.1483x over previous
import functools

import jax
import jax.numpy as jnp
from jax import lax
from jax.experimental import pallas as pl
from jax.experimental.pallas import tpu as pltpu
from jax.experimental.pallas import tpu_sc as plsc

VOCAB = 1000000
EMBED_DIM = 32
PADDING_IDX = 0

NUM_CORES = 2
NUM_SUBCORES = 16
NUM_WORKERS = NUM_CORES * NUM_SUBCORES

BATCH, SEQ = 4096, 200
N = BATCH * SEQ
PER_WORKER = N // NUM_WORKERS
CHUNK = 128
NCHUNK = PER_WORKER // CHUNK


NBUF = 8
LEAD = 4


def _fixup_chunk(idx_v, rows_b, g):
    zeros16 = jnp.zeros((16,), jnp.float32)
    macc = idx_v[g, pl.ds(0, 16)] == PADDING_IDX
    for l in range(1, CHUNK // 16):
        macc = jnp.logical_or(macc, idx_v[g, pl.ds(l * 16, 16)] == PADDING_IDX)

    @pl.when(jnp.any(macc))
    def _fixup():
        for gi in range(CHUNK // 16):
            iv = idx_v[g, pl.ds(gi * 16, 16)]
            m = iv == PADDING_IDX

            @pl.when(jnp.any(m))
            def _zgroup():
                rows_idx = lax.iota(jnp.int32, 16) + gi * 16
                for col in range(EMBED_DIM):
                    plsc.store_scatter(
                        rows_b,
                        [rows_idx, jnp.full((16,), col, jnp.int32)],
                        zeros16,
                        mask=m,
                    )


def _emb_body(idx_hbm, w_hbm, out_hbm, idx_v, rows, gsems, osems):
    wid = lax.axis_index("s") * NUM_CORES + lax.axis_index("c")
    cbase = wid * NCHUNK
    rbase = wid * PER_WORKER

    pltpu.sync_copy(idx_hbm.at[pl.ds(cbase, NCHUNK)], idx_v)

    def gather(g, b):
        pltpu.async_copy(w_hbm.at[idx_v.at[g]], rows[b], gsems[b])

    def gather_wait(g, b):
        pltpu.make_async_copy(w_hbm.at[idx_v.at[g]], rows[b], gsems[b]).wait()

    def out_copy(g, b):
        pltpu.async_copy(
            rows[b], out_hbm.at[pl.ds(rbase + g * CHUNK, CHUNK)], osems[b]
        )

    def out_wait(g, b):
        pltpu.make_async_copy(
            rows[b], out_hbm.at[pl.ds(rbase + g * CHUNK, CHUNK)], osems[b]
        ).wait()

    for g in range(LEAD):
        gather(g, g % NBUF)

    def step(t, carry):
        for b in range(NBUF):
            g = t * NBUF + b
            bl = (b + LEAD) % NBUF

            @pl.when(jnp.logical_and(g + LEAD >= NBUF, g + LEAD < NCHUNK))
            def _recycle():
                out_wait(g + LEAD - NBUF, bl)

            @pl.when(g + LEAD < NCHUNK)
            def _prefetch():
                gather(g + LEAD, bl)

            gather_wait(g, b)
            _fixup_chunk(idx_v, rows[b], g)
            out_copy(g, b)
        return carry

    lax.fori_loop(0, NCHUNK // NBUF, step, 0)

    for g in range(NCHUNK - NBUF, NCHUNK):
        out_wait(g, g % NBUF)


@jax.jit
def _emb_call(idx2d, weight):
    mesh = plsc.VectorSubcoreMesh(core_axis_name="c", subcore_axis_name="s")
    fn = functools.partial(
        pl.kernel,
        mesh=mesh,
        out_type=jax.ShapeDtypeStruct((N, EMBED_DIM), jnp.float32),
        scratch_types=[
            pltpu.VMEM((NCHUNK, CHUNK), jnp.int32),
            [pltpu.VMEM((CHUNK, EMBED_DIM), jnp.float32) for _ in range(NBUF)],
            [pltpu.SemaphoreType.DMA for _ in range(NBUF)],
            [pltpu.SemaphoreType.DMA for _ in range(NBUF)],
        ],
        compiler_params=pltpu.CompilerParams(
            needs_layout_passes=False, use_tc_tiling_on_sc=False
        ),
    )(_emb_body)
    return fn(idx2d, weight)


def kernel(input_ids, weight):
    idx2d = input_ids.astype(jnp.int32).reshape(NUM_WORKERS * NCHUNK, CHUNK)
    out = _emb_call(idx2d, weight)
    return out.reshape(BATCH, SEQ, EMBED_DIM)
